# baseline (speedup 1.0000x reference)
import functools

import jax
import jax.numpy as jnp
from jax import lax
from jax.experimental import pallas as pl
from jax.experimental.pallas import tpu as pltpu
from jax.experimental.pallas import tpu_sc as plsc

N = 1600000
D = 16
G = 1024

BLKL = 131072
GRID1 = -(-N // BLKL)

NC = 2
NS = 16
NW = NC * NS
ROWS_PER_TILE = N // NW
CHUNK = 10000
NCHUNK = ROWS_PER_TILE // CHUNK
ACC = NS * G
NHALF = 1


_LOG2E = 1.4426950408889634
_LN2 = 0.6931471805599453


def _bce_rowsum_body(src_ref, tgt_ref, out_ref):
    x = src_ref[...]
    t = tgt_ref[...]
    sp = _LN2 * jnp.log2(1.0 + jnp.exp2(jnp.abs(x) * (-_LOG2E)))
    bce = jnp.maximum(x, 0.0) - x * t + sp
    ones = jnp.ones((1, D), jnp.float32)
    r = lax.dot_general(ones, bce, (((1,), (0,)), ((), ())),
                        preferred_element_type=jnp.float32)
    out_ref[...] = r.reshape(BLKL)


def _bce_rowsums(src, target):
    src_t = jnp.transpose(src)
    tgt_t = jnp.transpose(target)
    return pl.pallas_call(
        _bce_rowsum_body,
        grid=(GRID1,),
        in_specs=[pl.BlockSpec((D, BLKL), lambda i: (0, i)),
                  pl.BlockSpec((D, BLKL), lambda i: (0, i))],
        out_specs=pl.BlockSpec((BLKL,), lambda i: (i,)),
        out_shape=jax.ShapeDtypeStruct((N,), jnp.float32),
    )(src_t, tgt_t)


def _zero_acc(acc_v):
    zeros16 = jnp.zeros((16,), jnp.float32)

    def zero_body(j, _):
        acc_v[j >> 3, pl.ds((j & 7) * 16, 16)] = zeros16
        return 0
    lax.fori_loop(0, ACC // 16, zero_body, 0, unroll=8)


def _sc_sums_body(r_hbm, idx_hbm, sacc_hbm, r_v0, r_v1, i_v0, i_v1,
                  sacc_v, sems):
    r_bufs = (r_v0, r_v1)
    i_bufs = (i_v0, i_v1)
    wid = lax.axis_index("c") * NS + lax.axis_index("s")
    base = wid * ROWS_PER_TILE
    lane = lax.iota(jnp.int32, 16)

    _zero_acc(sacc_v)

    def start(k):
        b = k & 1
        off = base + k * CHUNK
        return (pltpu.async_copy(r_hbm.at[pl.ds(off, CHUNK)], r_bufs[b],
                                 sems.at[b, 0]),
                pltpu.async_copy(idx_hbm.at[pl.ds(off, CHUNK)], i_bufs[b],
                                 sems.at[b, 1]))

    pend = start(0)
    for k in range(NCHUNK):
        nxt = start(k + 1) if k + 1 < NCHUNK else None
        pend[0].wait()
        pend[1].wait()
        r_v = r_bufs[k & 1]
        i_v = i_bufs[k & 1]

        def body(j, _):
            i16 = i_v[pl.ds(j * 16, 16)]
            r16 = r_v[pl.ds(j * 16, 16)]
            row = lax.shift_right_logical(i16, 3)
            col = (i16 & 7) * 16 + lane
            plsc.addupdate_scatter(sacc_v, [row, col], r16)
            return 0
        lax.fori_loop(0, CHUNK // 16, body, 0, unroll=8)
        pend = nxt

    pltpu.sync_copy(sacc_v, sacc_hbm.at[wid])


def _sc_counts_body(idx_hbm, cacc_hbm, i_v0, i_v1, cacc_v, sems):
    i_bufs = (i_v0, i_v1)
    wid = lax.axis_index("c") * NS + lax.axis_index("s")
    base = wid * ROWS_PER_TILE
    lane = lax.iota(jnp.int32, 16)
    ones16 = jnp.ones((16,), jnp.float32)

    _zero_acc(cacc_v)

    def start(k):
        b = k & 1
        return pltpu.async_copy(idx_hbm.at[pl.ds(base + k * CHUNK, CHUNK)],
                                i_bufs[b], sems.at[b])

    pend = start(0)
    for k in range(NCHUNK):
        nxt = start(k + 1) if k + 1 < NCHUNK else None
        pend.wait()
        i_v = i_bufs[k & 1]

        def body(j, _):
            i16 = i_v[pl.ds(j * 16, 16)]
            row = lax.shift_right_logical(i16, 3)
            col = (i16 & 7) * 16 + lane
            plsc.addupdate_scatter(cacc_v, [row, col], ones16)
            return 0
        lax.fori_loop(0, CHUNK // 16, body, 0, unroll=8)
        pend = nxt

    pltpu.sync_copy(cacc_v, cacc_hbm.at[wid])


def _sc_segment_sums(r_flat, index):
    mesh = plsc.VectorSubcoreMesh(core_axis_name="c", subcore_axis_name="s")
    f = pl.kernel(
        _sc_sums_body,
        out_type=jax.ShapeDtypeStruct((NW, 128, 128), jnp.float32),
        mesh=mesh,
        scratch_types=[
            pltpu.VMEM((CHUNK,), jnp.float32),
            pltpu.VMEM((CHUNK,), jnp.float32),
            pltpu.VMEM((CHUNK,), jnp.int32),
            pltpu.VMEM((CHUNK,), jnp.int32),
            pltpu.VMEM((128, 128), jnp.float32),
            pltpu.SemaphoreType.DMA((2, 2)),
        ],
        compiler_params=pltpu.CompilerParams(needs_layout_passes=False),
    )
    return f(r_flat, index)


def _sc_segment_counts(index):
    mesh = plsc.VectorSubcoreMesh(core_axis_name="c", subcore_axis_name="s")
    f = pl.kernel(
        _sc_counts_body,
        out_type=jax.ShapeDtypeStruct((NW, 128, 128), jnp.float32),
        mesh=mesh,
        scratch_types=[
            pltpu.VMEM((CHUNK,), jnp.int32),
            pltpu.VMEM((CHUNK,), jnp.int32),
            pltpu.VMEM((128, 128), jnp.float32),
            pltpu.SemaphoreType.DMA((2,)),
        ],
        compiler_params=pltpu.CompilerParams(needs_layout_passes=False),
    )
    return f(index)


def _finalize_body(s_ref, c_ref, out_ref):
    s = jnp.sum(s_ref[...].reshape(NW, 128, 128), axis=0)
    c = jnp.sum(c_ref[...].reshape(NW, 128, 128), axis=0)
    e = (lax.broadcasted_iota(jnp.int32, (128, 8), 0) // 16
         == lax.broadcasted_iota(jnp.int32, (128, 8), 1)).astype(jnp.float32)
    s2 = lax.dot_general(s, e, (((1,), (0,)), ((), ())),
                         preferred_element_type=jnp.float32)
    c2 = lax.dot_general(c, e, (((1,), (0,)), ((), ())),
                         preferred_element_type=jnp.float32)
    total = jnp.sum(s2 / jnp.maximum(c2, 1.0))
    gi = (lax.broadcasted_iota(jnp.int32, (128, 8), 0) * 8
          + lax.broadcasted_iota(jnp.int32, (128, 8), 1))
    nof = jnp.max(jnp.where(c2 > 0.0, gi + 1, 0))
    ans = total / (nof * D).astype(jnp.float32)
    out_ref[...] = jnp.broadcast_to(ans, (1, 1))


def _finalize(sacc, cacc):
    out = pl.pallas_call(
        _finalize_body,
        out_shape=jax.ShapeDtypeStruct((1, 1), jnp.float32),
    )(sacc.reshape(NW * 128, 128), cacc.reshape(NW * 128, 128))
    return out.reshape(())


def kernel(src, index, target):
    cacc = _sc_segment_counts(index)
    r_flat = _bce_rowsums(src, target)
    sacc = _sc_segment_sums(r_flat, index)
    return _finalize(sacc, cacc)


# speedup vs baseline: 1.2339x; 1.2339x over previous
import functools

import jax
import jax.numpy as jnp
from jax import lax
from jax.experimental import pallas as pl
from jax.experimental.pallas import tpu as pltpu
from jax.experimental.pallas import tpu_sc as plsc

N = 1600000
D = 16
G = 1024

BLKL = 163840
GRID1 = -(-N // BLKL)

NC = 2
NS = 16
NW = NC * NS
ROWS_PER_TILE = N // NW
CHUNK = 10000
NCHUNK = ROWS_PER_TILE // CHUNK
ACC = NS * G
NHALF = 1


_LOG2E = 1.4426950408889634
_LN2 = 0.6931471805599453


def _bce_rowsum_body(src_ref, tgt_ref, out_ref):
    x = src_ref[...]
    t = tgt_ref[...]
    sp = _LN2 * jnp.log2(1.0 + jnp.exp2(jnp.abs(x) * (-_LOG2E)))
    bce = jnp.maximum(x, 0.0) - x * t + sp
    ones = jnp.ones((1, D), jnp.float32)
    r = lax.dot_general(ones, bce, (((1,), (0,)), ((), ())),
                        preferred_element_type=jnp.float32)
    out_ref[...] = r.reshape(BLKL)


def _bce_rowsums(src, target):
    src_t = jnp.transpose(src)
    tgt_t = jnp.transpose(target)
    return pl.pallas_call(
        _bce_rowsum_body,
        grid=(GRID1,),
        in_specs=[pl.BlockSpec((D, BLKL), lambda i: (0, i)),
                  pl.BlockSpec((D, BLKL), lambda i: (0, i))],
        out_specs=pl.BlockSpec((BLKL,), lambda i: (i,)),
        out_shape=jax.ShapeDtypeStruct((N,), jnp.float32),
    )(src_t, tgt_t)


def _zero_acc(acc_v):
    zeros16 = jnp.zeros((16,), jnp.float32)

    def zero_body(j, _):
        acc_v[j >> 3, pl.ds((j & 7) * 16, 16)] = zeros16
        return 0
    lax.fori_loop(0, ACC // 16, zero_body, 0, unroll=8)


def _sc_sums_body(r_hbm, idx_hbm, sacc_hbm, cacc_hbm, r_v0, r_v1, i_v0, i_v1,
                  sacc_v, cacc_v, sems):
    r_bufs = (r_v0, r_v1)
    i_bufs = (i_v0, i_v1)
    wid = lax.axis_index("c") * NS + lax.axis_index("s")
    base = wid * ROWS_PER_TILE
    lane = lax.iota(jnp.int32, 16)
    ones16 = jnp.ones((16,), jnp.float32)

    _zero_acc(sacc_v)
    _zero_acc(cacc_v)

    def start(k):
        b = k & 1
        off = base + k * CHUNK
        return (pltpu.async_copy(r_hbm.at[pl.ds(off, CHUNK)], r_bufs[b],
                                 sems.at[b, 0]),
                pltpu.async_copy(idx_hbm.at[pl.ds(off, CHUNK)], i_bufs[b],
                                 sems.at[b, 1]))

    pend = start(0)
    for k in range(NCHUNK):
        nxt = start(k + 1) if k + 1 < NCHUNK else None
        pend[0].wait()
        pend[1].wait()
        r_v = r_bufs[k & 1]
        i_v = i_bufs[k & 1]

        def body(j, _):
            i16 = i_v[pl.ds(j * 16, 16)]
            r16 = r_v[pl.ds(j * 16, 16)]
            row = lax.shift_right_logical(i16, 3)
            col = (i16 & 7) * 16 + lane
            plsc.addupdate_scatter(sacc_v, [row, col], r16)
            plsc.addupdate_scatter(cacc_v, [row, col], ones16)
            return 0
        lax.fori_loop(0, CHUNK // 16, body, 0, unroll=8)
        pend = nxt

    pltpu.sync_copy(sacc_v, sacc_hbm.at[wid])
    pltpu.sync_copy(cacc_v, cacc_hbm.at[wid])


def _sc_segment_sums(r_flat, index):
    mesh = plsc.VectorSubcoreMesh(core_axis_name="c", subcore_axis_name="s")
    f = pl.kernel(
        _sc_sums_body,
        out_type=(jax.ShapeDtypeStruct((NW, 128, 128), jnp.float32),
                  jax.ShapeDtypeStruct((NW, 128, 128), jnp.float32)),
        mesh=mesh,
        scratch_types=[
            pltpu.VMEM((CHUNK,), jnp.float32),
            pltpu.VMEM((CHUNK,), jnp.float32),
            pltpu.VMEM((CHUNK,), jnp.int32),
            pltpu.VMEM((CHUNK,), jnp.int32),
            pltpu.VMEM((128, 128), jnp.float32),
            pltpu.VMEM((128, 128), jnp.float32),
            pltpu.SemaphoreType.DMA((2, 2)),
        ],
        compiler_params=pltpu.CompilerParams(needs_layout_passes=False),
    )
    return f(r_flat, index)


def _finalize_body(s_ref, c_ref, out_ref):
    s = jnp.sum(s_ref[...].reshape(NW, 128, 128), axis=0)
    c = jnp.sum(c_ref[...].reshape(NW, 128, 128), axis=0)
    e = (lax.broadcasted_iota(jnp.int32, (128, 8), 0) // 16
         == lax.broadcasted_iota(jnp.int32, (128, 8), 1)).astype(jnp.float32)
    s2 = lax.dot_general(s, e, (((1,), (0,)), ((), ())),
                         preferred_element_type=jnp.float32)
    c2 = lax.dot_general(c, e, (((1,), (0,)), ((), ())),
                         preferred_element_type=jnp.float32)
    total = jnp.sum(s2 / jnp.maximum(c2, 1.0))
    gi = (lax.broadcasted_iota(jnp.int32, (128, 8), 0) * 8
          + lax.broadcasted_iota(jnp.int32, (128, 8), 1))
    nof = jnp.max(jnp.where(c2 > 0.0, gi + 1, 0))
    ans = total / (nof * D).astype(jnp.float32)
    out_ref[...] = jnp.broadcast_to(ans, (1, 1))


def _finalize(sacc, cacc):
    out = pl.pallas_call(
        _finalize_body,
        out_shape=jax.ShapeDtypeStruct((1, 1), jnp.float32),
    )(sacc.reshape(NW * 128, 128), cacc.reshape(NW * 128, 128))
    return out.reshape(())


def kernel(src, index, target):
    r_flat = _bce_rowsums(src, target)
    sacc, cacc = _sc_segment_sums(r_flat, index)
    return _finalize(sacc, cacc)
